# baseline (speedup 1.0000x reference)
import functools

import jax
import jax.numpy as jnp
from jax import lax
from jax.experimental import pallas as pl
from jax.experimental.pallas import tpu as pltpu
from jax.experimental.pallas import tpu_sc as plsc


def kernel(input_tensor, table):
    B, H = input_tensor.shape
    V, D = table.shape
    N = B * H
    Hp = (H + 7) // 8 * 8
    Dp = 128

    info = plsc.get_sparse_core_info()
    NC, NS = info.num_cores, info.num_subcores
    NW = NC * NS

    K = 2 * H
    assert N % (NW * K) == 0
    n_per_w = N // NW
    n_ck = n_per_w // K
    b_per_w = B // NW

    KG = 104
    idxT = jnp.pad(input_tensor.T.astype(jnp.int32), ((0, Hp - H), (0, 0)))

    mesh = plsc.VectorSubcoreMesh(core_axis_name="c", subcore_axis_name="s")

    NBUF = 4
    DEPTH = 2

    @functools.partial(
        pl.kernel,
        out_type=jax.ShapeDtypeStruct((B, Hp, Dp), jnp.float32),
        mesh=mesh,
        scratch_types=[
            pltpu.VMEM((Hp, b_per_w), jnp.int32),
            pltpu.VMEM((n_ck * KG,), jnp.int32),
            pltpu.VMEM((NBUF, KG, D), jnp.float32),
            pltpu.SemaphoreType.DMA,
            [pltpu.SemaphoreType.DMA] * NBUF,
            [pltpu.SemaphoreType.DMA] * NBUF,
        ],
        compiler_params=pltpu.CompilerParams(
            use_tc_tiling_on_sc=False, needs_layout_passes=False
        ),
    )
    def emb(idx_hbm, table_hbm, out_hbm, idxT_v, idx_c, rows_v, isem, gsems, wsems):
        wid = lax.axis_index("s") * NC + lax.axis_index("c")
        b0 = wid * b_per_w
        pltpu.async_copy(
            idx_hbm.at[pl.ds(0, Hp), pl.ds(b0, b_per_w)], idxT_v, isem
        ).wait()

        lane = lax.iota(jnp.int32, 16)
        lane_off = (lane >> 1) * KG + (lane & 1) * H

        h_vec = jnp.zeros((16,), jnp.int32)

        def trow(h, carry):
            for k in range(b_per_w // 16):
                v = plsc.load_gather(idxT_v, [h_vec + h, lane + 16 * k])
                base = (8 * k) * KG + h
                plsc.store_scatter(idx_c, [lane_off + base], v)
            return carry

        lax.fori_loop(0, H, trow, 0)

        dup_mask = lane < (KG - K)

        def dfill(c, carry):
            v = idx_c[pl.ds(c * KG, 16)]
            plsc.store_scatter(idx_c, [c * KG + K + lane], v, mask=dup_mask)
            return carry

        lax.fori_loop(0, n_ck, dfill, 0)

        def gstart(c, j):
            pltpu.async_copy(
                table_hbm.at[idx_c.at[pl.ds(c * KG, KG)]], rows_v.at[j], gsems[j]
            )

        def gwait(c, j):
            pltpu.make_async_copy(
                table_hbm.at[idx_c.at[pl.ds(c * KG, KG)]], rows_v.at[j], gsems[j]
            ).wait()

        def wstart(c, j):
            b = b0 + 2 * c
            pltpu.async_copy(
                rows_v.at[j, pl.ds(0, H)],
                out_hbm.at[b, pl.ds(0, H), pl.ds(0, D)],
                wsems[j],
            )
            pltpu.async_copy(
                rows_v.at[j, pl.ds(H, H)],
                out_hbm.at[b + 1, pl.ds(0, H), pl.ds(0, D)],
                wsems[j],
            )

        def wwait(c, j):
            b = b0 + 2 * c
            pltpu.make_async_copy(
                rows_v.at[j, pl.ds(0, H)],
                out_hbm.at[b, pl.ds(0, H), pl.ds(0, D)],
                wsems[j],
            ).wait()
            pltpu.make_async_copy(
                rows_v.at[j, pl.ds(H, H)],
                out_hbm.at[b + 1, pl.ds(0, H), pl.ds(0, D)],
                wsems[j],
            ).wait()

        for d in range(DEPTH):
            gstart(d, d)

        def body(gi, carry):
            base = gi * NBUF
            for j in range(NBUF):
                c = base + j
                jj = (j + DEPTH) % NBUF

                @pl.when(c >= NBUF - DEPTH)
                def _():
                    wwait(c - (NBUF - DEPTH), jj)

                @pl.when(c + DEPTH < n_ck)
                def _():
                    gstart(c + DEPTH, jj)

                gwait(c, j)
                wstart(c, j)
            return carry

        lax.fori_loop(0, n_ck // NBUF, body, 0)
        for c in range(n_ck - (NBUF - DEPTH), n_ck):
            wwait(c, c % NBUF)

    out_p = emb(idxT, table)
    return out_p[:, :H, :D]
